# baseline (speedup 1.0000x reference)
import functools

import jax
import jax.numpy as jnp
from jax import lax
from jax.experimental import pallas as pl
from jax.experimental.pallas import tpu as pltpu
from jax.experimental.pallas import tpu_sc as plsc

NC, NS = 2, 16
NW = NC * NS
BATCH = 16384
BPW = BATCH // NW
CH = 128
NCH = BPW // CH
NVIEWS = 1000000
NELEM = NVIEWS * 3
LW = 128
NWIN = (NELEM + LW - 1) // LW + 1
WMAX = NWIN - 1
GROUPS = CH // 16

_MESH = plsc.VectorSubcoreMesh(
    core_axis_name="c", subcore_axis_name="s",
    num_cores=NC, num_subcores=NS,
)


@functools.partial(
    pl.kernel,
    out_type=jax.ShapeDtypeStruct((NW, NCH, CH, 3), jnp.float32),
    mesh=_MESH,
    scratch_types=[
        pltpu.VMEM((NCH, CH), jnp.int32),
        pltpu.VMEM((2, CH), jnp.int32),
        pltpu.VMEM((CH,), jnp.int32),
        pltpu.VMEM((2, CH, LW), jnp.float32),
        pltpu.VMEM((NCH, CH, 3), jnp.float32),
        pltpu.SemaphoreType.DMA,
    ],
    compiler_params=pltpu.CompilerParams(needs_layout_passes=False),
)
def _gather_sc(t2_hbm, idx_hbm, out_hbm, idx_v, win_i, off_v, wins, rows, sem):
    wid = lax.axis_index("s") * NC + lax.axis_index("c")
    pltpu.sync_copy(idx_hbm.at[wid], idx_v)
    lanes = lax.iota(jnp.int32, 16)
    for k in range(NCH):
        for g in range(GROUPS):
            v = idx_v[k, pl.ds(g * 16, 16)]
            e = v * 3
            w0 = e >> 7
            win_i[0, pl.ds(g * 16, 16)] = w0
            win_i[1, pl.ds(g * 16, 16)] = jnp.minimum(w0 + 1, WMAX)
            off_v[pl.ds(g * 16, 16)] = e & 127
        c0 = pltpu.async_copy(t2_hbm.at[win_i.at[0]], wins.at[0], sem)
        c1 = pltpu.async_copy(t2_hbm.at[win_i.at[1]], wins.at[1], sem)
        c0.wait()
        c1.wait()
        kk = jnp.full((16,), k, jnp.int32)
        for g in range(GROUPS):
            i = lanes + (g * 16)
            o = off_v[pl.ds(g * 16, 16)]
            for c in range(3):
                p = o + c
                vals = plsc.load_gather(wins, [p >> 7, i, p & 127])
                plsc.store_scatter(rows, [kk, i, jnp.full((16,), c, jnp.int32)], vals)
    pltpu.sync_copy(rows, out_hbm.at[wid])


def kernel(view_correction, index):
    flat = view_correction.reshape(-1)
    t2 = jnp.pad(flat, (0, NWIN * LW - NELEM)).reshape(NWIN, LW)
    idx = index.reshape(NW, NCH, CH)
    out = _gather_sc(t2, idx)
    return out.reshape(BATCH, 3)
